# baseline (speedup 1.0000x reference)
import functools

import jax
import jax.numpy as jnp
from jax import lax
from jax.experimental import pallas as pl
from jax.experimental.pallas import tpu as pltpu
from jax.experimental.pallas import tpu_sc as plsc

SEQ, BATCH, VOCAB, EMB = 50, 4096, 100000, 64
TOT = SEQ * BATCH
CHUNK = 128
NC, NS = 2, 16
NW = NC * NS
PER_W = TOT // NW
IDXROWS = TOT // CHUNK
ROWS_W = PER_W // CHUNK

G = 2
SUPER = G * CHUNK
NSUP = ROWS_W // G
NBUF = 5
P = 3
NGRP = NSUP // NBUF

TB = 8
TROWS = BATCH // TB


def _sc_gather(idx2d, table):
    mesh = plsc.VectorSubcoreMesh(core_axis_name="c", subcore_axis_name="s")

    @functools.partial(
        pl.kernel,
        mesh=mesh,
        out_type=jax.ShapeDtypeStruct((TOT, EMB), jnp.float32),
        compiler_params=pltpu.CompilerParams(use_tc_tiling_on_sc=False),
        scratch_types=[pltpu.VMEM((ROWS_W, CHUNK), jnp.int32)]
        + [pltpu.VMEM((SUPER, EMB), jnp.float32) for _ in range(NBUF)]
        + [pltpu.SemaphoreType.DMA for _ in range(2 * NBUF)],
    )
    def k(table_hbm, idx_hbm, out_hbm, idx_v, *rest):
        bufs = list(rest[:NBUF])
        gsems = list(rest[NBUF:2 * NBUF])
        osems = list(rest[2 * NBUF:3 * NBUF])
        wid = lax.axis_index("s") * NC + lax.axis_index("c")
        pltpu.sync_copy(idx_hbm.at[pl.ds(wid * ROWS_W, ROWS_W)], idx_v)
        row0 = wid * PER_W

        def fire_g(s, b):
            for c in range(G):
                pltpu.make_async_copy(
                    table_hbm.at[idx_v.at[s * G + c]],
                    bufs[b].at[pl.ds(c * CHUNK, CHUNK)],
                    gsems[b]).start()

        def wait_g(b):
            for c in range(G):
                pltpu.make_async_copy(
                    table_hbm.at[idx_v.at[0]],
                    bufs[b].at[pl.ds(c * CHUNK, CHUNK)],
                    gsems[b]).wait()

        def fire_o(s, b):
            pltpu.make_async_copy(
                bufs[b],
                out_hbm.at[pl.ds(row0 + s * SUPER, SUPER)],
                osems[b]).start()

        def wait_o(b):
            pltpu.make_async_copy(
                bufs[b],
                out_hbm.at[pl.ds(0, SUPER)],
                osems[b]).wait()

        for s in range(P):
            fire_g(s, s % NBUF)

        for b in range(NBUF):
            sp = b + P
            bp = sp % NBUF
            if sp >= NBUF:
                wait_o(bp)
            fire_g(sp, bp)
            wait_g(b)
            fire_o(b, b)

        def grp(go, carry):
            for b in range(NBUF):
                s = go * NBUF + b
                bp = (b + P) % NBUF
                wait_o(bp)
                fire_g(s + P, bp)
                wait_g(b)
                fire_o(s, b)
            return carry

        lax.fori_loop(1, NGRP - 1, grp, 0)

        for b in range(NBUF):
            s = (NGRP - 1) * NBUF + b
            sp = s + P
            bp = (b + P) % NBUF
            if sp < NSUP:
                wait_o(bp)
                fire_g(sp, bp)
            wait_g(b)
            fire_o(s, b)

        for b in range(NBUF):
            wait_o(b)

    return k(table, idx2d)


def _tc_mask_and_idx(idx):
    def mk(idx_ref, mask_ref, lin_ref):
        x = idx_ref[...]
        mask_ref[...] = (x == 0).astype(jnp.float32)
        lin_ref[...] = x.reshape(IDXROWS, CHUNK)

    return pl.pallas_call(
        mk,
        out_shape=(
            jax.ShapeDtypeStruct((SEQ, BATCH), jnp.float32),
            jax.ShapeDtypeStruct((IDXROWS, CHUNK), jnp.int32),
        ),
    )(idx)


def _tc_transpose(pairs):
    def tk(in_ref, out_ref):
        x = in_ref[...].reshape(TROWS // 2, 2, EMB)
        out_ref[...] = x.transpose(2, 0, 1).reshape(1, EMB, TROWS)

    return pl.pallas_call(
        tk,
        grid=(SEQ, TB),
        in_specs=[
            pl.BlockSpec((TROWS // 2, 2 * EMB), lambda s, j: (s * TB + j, 0))
        ],
        out_specs=pl.BlockSpec((1, EMB, TROWS), lambda s, j: (s, 0, j)),
        out_shape=jax.ShapeDtypeStruct((SEQ, EMB, BATCH), jnp.float32),
    )(pairs)


def kernel(input_idx, word_lut):
    idx32 = input_idx.astype(jnp.int32)
    mask, idx2d = _tc_mask_and_idx(idx32)
    rows = _sc_gather(idx2d, word_lut)
    emb = _tc_transpose(rows.reshape(TOT // 2, 2 * EMB)).swapaxes(1, 2)
    return emb, mask.T


# speedup vs baseline: 8.4184x; 8.4184x over previous
import functools

import jax
import jax.numpy as jnp
from jax import lax
from jax.experimental import pallas as pl
from jax.experimental.pallas import tpu as pltpu
from jax.experimental.pallas import tpu_sc as plsc

SEQ, BATCH, VOCAB, EMB = 50, 4096, 100000, 64
TOT = SEQ * BATCH
CHUNK = 128
NC, NS = 2, 16
NW = NC * NS
PER_W = TOT // NW
CHUNKS_PER_W = PER_W // CHUNK

G = 2
SUPER = G * CHUNK
NSUP = CHUNKS_PER_W // G
NBUF = 5
P = 4
NGRP = NSUP // NBUF


def _sc_gather(idx1d, table):
    mesh = plsc.VectorSubcoreMesh(core_axis_name="c", subcore_axis_name="s")

    @functools.partial(
        pl.kernel,
        mesh=mesh,
        out_type=jax.ShapeDtypeStruct((SEQ, BATCH, EMB), jnp.float32),
        compiler_params=pltpu.CompilerParams(use_tc_tiling_on_sc=False),
        scratch_types=[pltpu.VMEM((PER_W,), jnp.int32)]
        + [pltpu.VMEM((SUPER, EMB), jnp.float32) for _ in range(NBUF)]
        + [pltpu.SemaphoreType.DMA for _ in range(2 * NBUF)],
    )
    def k(table_hbm, idx_hbm, out_hbm, idx_v, *rest):
        bufs = list(rest[:NBUF])
        gsems = list(rest[NBUF:2 * NBUF])
        osems = list(rest[2 * NBUF:3 * NBUF])
        wid = lax.axis_index("s") * NC + lax.axis_index("c")
        pltpu.sync_copy(idx_hbm.at[pl.ds(wid * PER_W, PER_W)], idx_v)
        unit0 = wid * NSUP

        def fire_g(s, b):
            for c in range(G):
                pltpu.make_async_copy(
                    table_hbm.at[idx_v.at[pl.ds((s * G + c) * CHUNK, CHUNK)]],
                    bufs[b].at[pl.ds(c * CHUNK, CHUNK)],
                    gsems[b]).start()

        def wait_g(b):
            for c in range(G):
                pltpu.make_async_copy(
                    table_hbm.at[idx_v.at[pl.ds(0, CHUNK)]],
                    bufs[b].at[pl.ds(c * CHUNK, CHUNK)],
                    gsems[b]).wait()

        NB = BATCH // SUPER

        def fire_o(s, b):
            g = unit0 + s
            pltpu.make_async_copy(
                bufs[b],
                out_hbm.at[g // NB, pl.ds((g % NB) * SUPER, SUPER)],
                osems[b]).start()

        def wait_o(b):
            pltpu.make_async_copy(
                bufs[b],
                out_hbm.at[0, pl.ds(0, SUPER)],
                osems[b]).wait()

        for s in range(P):
            fire_g(s, s % NBUF)

        for b in range(NBUF):
            sp = b + P
            bp = sp % NBUF
            if sp >= NBUF:
                wait_o(bp)
            fire_g(sp, bp)
            wait_g(b)
            fire_o(b, b)

        def grp(go, carry):
            for b in range(NBUF):
                s = go * NBUF + b
                bp = (b + P) % NBUF
                wait_o(bp)
                fire_g(s + P, bp)
                wait_g(b)
                fire_o(s, b)
            return carry

        lax.fori_loop(1, NGRP - 1, grp, 0)

        for b in range(NBUF):
            s = (NGRP - 1) * NBUF + b
            sp = s + P
            bp = (b + P) % NBUF
            if sp < NSUP:
                wait_o(bp)
                fire_g(sp, bp)
            wait_g(b)
            fire_o(s, b)

        for b in range(NBUF):
            wait_o(b)

    return k(table, idx1d)


def _tc_mask(idx):
    def mk(idx_ref, out_ref):
        out_ref[...] = (idx_ref[...] == 0).astype(jnp.float32)

    return pl.pallas_call(
        mk,
        out_shape=jax.ShapeDtypeStruct((SEQ, BATCH), jnp.float32),
    )(idx)


def kernel(input_idx, word_lut):
    idx32 = input_idx.astype(jnp.int32)
    emb = _sc_gather(idx32.reshape(-1), word_lut)
    mask = _tc_mask(idx32).T
    return emb, mask
